# baseline (speedup 1.0000x reference)
import functools

import jax
import jax.numpy as jnp
from jax import lax
from jax.experimental import pallas as pl
from jax.experimental.pallas import tpu as pltpu
from jax.experimental.pallas import tpu_sc as plsc

NUM_SUBCORES = 16
LANES = 16
NSEG = 16
NHEAD = 9
LN2 = 0.6931471805599453


def _dense_body(r, x_ref, wa_ref, wd_ref, ba_ref, bd_ref, out_ref):
    x = x_ref[...]
    dims = (((0,), (1,)), ((), ()))
    ya = lax.dot_general(wa_ref[...], x, dims,
                         preferred_element_type=jnp.float32)
    ya = ya + ba_ref[...].reshape(8, 1)
    yd = lax.dot_general(wd_ref[...], x, dims,
                         preferred_element_type=jnp.float32)
    yd = yd + bd_ref[...].reshape(1, 1)
    amax = jnp.max(ya, axis=0, keepdims=True)
    lse = jnp.log(jnp.sum(jnp.exp(ya - amax), axis=0, keepdims=True)) + amax
    z = jnp.concatenate([ya - lse, yd], axis=0)
    out_ref[...] = z.reshape(NHEAD * r)


def _dense_call(x, w_act, w_dev, b_act, b_dev, block_rows):
    n, e = x.shape
    a = w_act.shape[1]
    return pl.pallas_call(
        functools.partial(_dense_body, block_rows),
        grid=(n // block_rows,),
        in_specs=[
            pl.BlockSpec((block_rows, e), lambda i: (i, 0)),
            pl.BlockSpec((e, a), lambda i: (0, 0)),
            pl.BlockSpec((e, 1), lambda i: (0, 0)),
            pl.BlockSpec((a,), lambda i: (0,)),
            pl.BlockSpec((1,), lambda i: (0,)),
        ],
        out_specs=pl.BlockSpec((NHEAD * block_rows,), lambda i: (i,)),
        out_shape=jax.ShapeDtypeStruct((NHEAD * n,), jnp.float32),
    )(x, w_act, w_dev, b_act, b_dev)


def _sc_body(chunk, dense_block, main_hbm, idx_hbm, clse_hbm,
             dv, iv, lm_ref, s_ref, shared, part, clse_ref):
    minf = jnp.float32(-jnp.inf)
    wid = lax.axis_index("s")
    per = dense_block // chunk
    sbase = (wid // per) * NHEAD * dense_block + (wid % per) * chunk
    pltpu.sync_copy(main_hbm.at[pl.ds(sbase + 8 * dense_block, chunk)], dv)
    pltpu.sync_copy(idx_hbm.at[pl.ds(wid * chunk, chunk)], iv)

    nvec = chunk // LANES
    lane = lax.iota(jnp.int32, LANES)

    s_ref[...] = jnp.zeros((LANES,), jnp.float32)

    first = iv[pl.ds(0, LANES)][0]
    last = iv[pl.ds(chunk - LANES, LANES)][LANES - 1]

    @pl.when(first == last)
    def _chunk_uniform():
        def body(i, acc):
            return jnp.maximum(acc, dv[pl.ds(i * LANES, LANES)])

        acc = lax.fori_loop(0, nvec, body, jnp.full((LANES,), minf, jnp.float32),
                            unroll=8)
        m = jnp.max(acc)
        lm_ref[...] = jnp.where(lane == first, m, minf)

        def body2u(i, acc):
            return acc + jnp.exp(dv[pl.ds(i * LANES, LANES)] - m)

        acc2 = lax.fori_loop(0, nvec, body2u, jnp.zeros((LANES,), jnp.float32),
                             unroll=8)
        s_ref[...] = jnp.where(lane == first, jnp.sum(acc2), 0.0)

    @pl.when(first != last)
    def _chunk_mixed():
        def body(i, lm):
            v = dv[pl.ds(i * LANES, LANES)]
            sg = iv[pl.ds(i * LANES, LANES)]
            s0 = sg[0]
            s15 = sg[LANES - 1]

            def vec_uniform(lm):
                return jnp.where(lane == s0, jnp.maximum(lm, jnp.max(v)), lm)

            def vec_mixed(lm):
                def seg_loop(b, lm):
                    mb = jnp.max(jnp.where(sg == b, v, minf))
                    return jnp.where(lane == b, jnp.maximum(lm, mb), lm)

                return lax.fori_loop(0, NSEG, seg_loop, lm)

            return lax.cond(s0 == s15, vec_uniform, vec_mixed, lm)

        lm_ref[...] = lax.fori_loop(
            0, nvec, body, jnp.full((LANES,), minf, jnp.float32))

    @pl.when(first != last)
    def _chunk_mixed2():
        def body2(i, c):
            v = dv[pl.ds(i * LANES, LANES)]
            sg = iv[pl.ds(i * LANES, LANES)]
            shift = plsc.load_gather(lm_ref, [sg])
            plsc.addupdate_scatter(s_ref, [sg], jnp.exp(v - shift))
            return c

        lax.fori_loop(0, nvec, body2, 0, unroll=8)

    pltpu.sync_copy(lm_ref, shared.at[pl.ds(wid * 2 * LANES, LANES)])
    pltpu.sync_copy(s_ref, shared.at[pl.ds(wid * 2 * LANES + LANES, LANES)])
    plsc.subcore_barrier()

    @pl.when(wid == 0)
    def _reduce_and_emit():
        pltpu.sync_copy(shared, part)

        def mbody(w, m):
            return jnp.maximum(m, part[pl.ds(w * 2 * LANES, LANES)])

        m = lax.fori_loop(0, NUM_SUBCORES, mbody,
                          jnp.full((LANES,), minf, jnp.float32))

        def sbody(w, s):
            return s + part[pl.ds(w * 2 * LANES + LANES, LANES)] * jnp.exp(
                part[pl.ds(w * 2 * LANES, LANES)] - m)

        s = lax.fori_loop(0, NUM_SUBCORES, sbody,
                          jnp.zeros((LANES,), jnp.float32))

        bits = plsc.bitcast(s, jnp.int32)
        y = ((bits >> 23) - 127).astype(jnp.float32) * jnp.float32(LN2)
        for _ in range(4):
            y = y + s * jnp.exp(-y) - 1.0
        clse_ref[...] = m + y
        pltpu.sync_copy(clse_ref, clse_hbm)


def _sc_call(main_flat, idx, dense_block):
    n = idx.shape[0]
    chunk = n // NUM_SUBCORES
    mesh = plsc.VectorSubcoreMesh(
        core_axis_name="c", subcore_axis_name="s",
        num_cores=1, num_subcores=NUM_SUBCORES,
    )
    return pl.kernel(
        functools.partial(_sc_body, chunk, dense_block),
        out_type=jax.ShapeDtypeStruct((LANES,), jnp.float32),
        mesh=mesh,
        compiler_params=pltpu.CompilerParams(needs_layout_passes=False),
        scratch_types=[
            pltpu.VMEM((chunk,), jnp.float32),
            pltpu.VMEM((chunk,), jnp.int32),
            pltpu.VMEM((LANES,), jnp.float32),
            pltpu.VMEM((LANES,), jnp.float32),
            pltpu.VMEM_SHARED((NUM_SUBCORES * 2 * LANES,), jnp.float32),
            pltpu.VMEM((NUM_SUBCORES * 2 * LANES,), jnp.float32),
            pltpu.VMEM((LANES,), jnp.float32),
        ],
    )(main_flat, idx)


def _comb_body(r, main_ref, idx_ref, clse_ref, out_ref):
    z = main_ref[...].reshape(NHEAD, r)
    la3 = z[0:8, :].reshape(8, r // 128, 128)
    d2 = z[8:9, :].reshape(r // 128, 128)
    sg = idx_ref[...].reshape(r // 128, 128)
    clse = clse_ref[...]
    corr = jnp.zeros((r // 128, 128), jnp.float32)
    for b in range(NSEG):
        corr = jnp.where(sg == b, clse[b], corr)
    c2 = d2 - corr
    out_ref[...] = jnp.transpose(la3, (1, 0, 2)) + c2[:, None, :]


def _comb_call(main_flat, idx, clse, block_rows):
    n = idx.shape[0]
    t = n // 128
    bt = block_rows // 128
    return pl.pallas_call(
        functools.partial(_comb_body, block_rows),
        grid=(n // block_rows,),
        in_specs=[
            pl.BlockSpec((NHEAD * block_rows,), lambda i: (i,)),
            pl.BlockSpec((block_rows,), lambda i: (i,)),
            pl.BlockSpec((LANES,), lambda i: (0,)),
        ],
        out_specs=pl.BlockSpec((bt, 8, 128), lambda i: (i, 0, 0)),
        out_shape=jax.ShapeDtypeStruct((t, 8, 128), jnp.float32),
    )(main_flat, idx, clse)


def kernel(embed_states, batch_index, W_dev, b_dev, W_act, b_act):
    n, e = embed_states.shape
    a = W_act.shape[1]
    idx = batch_index.astype(jnp.int32)
    main = _dense_call(embed_states, W_act, W_dev, b_act, b_dev,
                       block_rows=4096)
    clse = _sc_call(main, idx, dense_block=4096)
    out3 = _comb_call(main, idx, clse, block_rows=4096)
    return out3.transpose(0, 2, 1).reshape(n, a)


# speedup vs baseline: 1.0275x; 1.0275x over previous
import functools

import jax
import jax.numpy as jnp
from jax import lax
from jax.experimental import pallas as pl
from jax.experimental.pallas import tpu as pltpu
from jax.experimental.pallas import tpu_sc as plsc

NUM_SUBCORES = 16
LANES = 16
NSEG = 16
NHEAD = 9
LN2 = 0.6931471805599453


def _dense_body(r, x_ref, wt_ref, bt_ref, out_ref):
    x = x_ref[...]
    yt = lax.dot_general(wt_ref[...], x, (((1,), (1,)), ((), ())),
                         preferred_element_type=jnp.float32)
    yt = yt + bt_ref[...]
    ya = yt[0:8, :]
    amax = jnp.max(ya, axis=0, keepdims=True)
    lse = jnp.log(jnp.sum(jnp.exp(ya - amax), axis=0, keepdims=True)) + amax
    z = jnp.concatenate([ya - lse, yt[8:9, :]], axis=0)
    out_ref[...] = z.reshape(NHEAD * r)


def _dense_call(x, w_t, b_t, block_rows):
    n, e = x.shape
    return pl.pallas_call(
        functools.partial(_dense_body, block_rows),
        grid=(n // block_rows,),
        in_specs=[
            pl.BlockSpec((block_rows, e), lambda i: (i, 0)),
            pl.BlockSpec((NHEAD, e), lambda i: (0, 0)),
            pl.BlockSpec((NHEAD, 1), lambda i: (0, 0)),
        ],
        out_specs=pl.BlockSpec((NHEAD * block_rows,), lambda i: (i,)),
        out_shape=jax.ShapeDtypeStruct((NHEAD * n,), jnp.float32),
    )(x, w_t, b_t)


def _sc_body(chunk, dense_block, main_hbm, idx_hbm, clse_hbm,
             dv, iv, lm_ref, s_ref, shared, part, clse_ref):
    minf = jnp.float32(-jnp.inf)
    wid = lax.axis_index("s")
    per = dense_block // chunk
    sbase = (wid // per) * NHEAD * dense_block + (wid % per) * chunk
    pltpu.sync_copy(main_hbm.at[pl.ds(sbase + 8 * dense_block, chunk)], dv)
    pltpu.sync_copy(idx_hbm.at[pl.ds(wid * chunk, chunk)], iv)

    nvec = chunk // LANES
    lane = lax.iota(jnp.int32, LANES)

    s_ref[...] = jnp.zeros((LANES,), jnp.float32)

    first = iv[pl.ds(0, LANES)][0]
    last = iv[pl.ds(chunk - LANES, LANES)][LANES - 1]

    @pl.when(first == last)
    def _chunk_uniform():
        def body(i, acc):
            return jnp.maximum(acc, dv[pl.ds(i * LANES, LANES)])

        acc = lax.fori_loop(0, nvec, body, jnp.full((LANES,), minf, jnp.float32),
                            unroll=8)
        m = jnp.max(acc)
        lm_ref[...] = jnp.where(lane == first, m, minf)

        def body2u(i, acc):
            return acc + jnp.exp(dv[pl.ds(i * LANES, LANES)] - m)

        acc2 = lax.fori_loop(0, nvec, body2u, jnp.zeros((LANES,), jnp.float32),
                             unroll=8)
        s_ref[...] = jnp.where(lane == first, jnp.sum(acc2), 0.0)

    @pl.when(first != last)
    def _chunk_mixed():
        def body(i, lm):
            v = dv[pl.ds(i * LANES, LANES)]
            sg = iv[pl.ds(i * LANES, LANES)]
            s0 = sg[0]
            s15 = sg[LANES - 1]

            def vec_uniform(lm):
                return jnp.where(lane == s0, jnp.maximum(lm, jnp.max(v)), lm)

            def vec_mixed(lm):
                def seg_loop(b, lm):
                    mb = jnp.max(jnp.where(sg == b, v, minf))
                    return jnp.where(lane == b, jnp.maximum(lm, mb), lm)

                return lax.fori_loop(0, NSEG, seg_loop, lm)

            return lax.cond(s0 == s15, vec_uniform, vec_mixed, lm)

        lm_ref[...] = lax.fori_loop(
            0, nvec, body, jnp.full((LANES,), minf, jnp.float32))

    @pl.when(first != last)
    def _chunk_mixed2():
        def body2(i, c):
            v = dv[pl.ds(i * LANES, LANES)]
            sg = iv[pl.ds(i * LANES, LANES)]
            shift = plsc.load_gather(lm_ref, [sg])
            plsc.addupdate_scatter(s_ref, [sg], jnp.exp(v - shift))
            return c

        lax.fori_loop(0, nvec, body2, 0, unroll=8)

    pltpu.sync_copy(lm_ref, shared.at[pl.ds(wid * 2 * LANES, LANES)])
    pltpu.sync_copy(s_ref, shared.at[pl.ds(wid * 2 * LANES + LANES, LANES)])
    plsc.subcore_barrier()

    @pl.when(wid == 0)
    def _reduce_and_emit():
        pltpu.sync_copy(shared, part)

        def mbody(w, m):
            return jnp.maximum(m, part[pl.ds(w * 2 * LANES, LANES)])

        m = lax.fori_loop(0, NUM_SUBCORES, mbody,
                          jnp.full((LANES,), minf, jnp.float32))

        def sbody(w, s):
            return s + part[pl.ds(w * 2 * LANES + LANES, LANES)] * jnp.exp(
                part[pl.ds(w * 2 * LANES, LANES)] - m)

        s = lax.fori_loop(0, NUM_SUBCORES, sbody,
                          jnp.zeros((LANES,), jnp.float32))

        bits = plsc.bitcast(s, jnp.int32)
        y = ((bits >> 23) - 127).astype(jnp.float32) * jnp.float32(LN2)
        for _ in range(4):
            y = y + s * jnp.exp(-y) - 1.0
        clse_ref[...] = m + y
        pltpu.sync_copy(clse_ref, clse_hbm)


def _sc_call(main_flat, idx, dense_block):
    n = idx.shape[0]
    chunk = n // NUM_SUBCORES
    mesh = plsc.VectorSubcoreMesh(
        core_axis_name="c", subcore_axis_name="s",
        num_cores=1, num_subcores=NUM_SUBCORES,
    )
    return pl.kernel(
        functools.partial(_sc_body, chunk, dense_block),
        out_type=jax.ShapeDtypeStruct((LANES,), jnp.float32),
        mesh=mesh,
        compiler_params=pltpu.CompilerParams(needs_layout_passes=False),
        scratch_types=[
            pltpu.VMEM((chunk,), jnp.float32),
            pltpu.VMEM((chunk,), jnp.int32),
            pltpu.VMEM((LANES,), jnp.float32),
            pltpu.VMEM((LANES,), jnp.float32),
            pltpu.VMEM_SHARED((NUM_SUBCORES * 2 * LANES,), jnp.float32),
            pltpu.VMEM((NUM_SUBCORES * 2 * LANES,), jnp.float32),
            pltpu.VMEM((LANES,), jnp.float32),
        ],
    )(main_flat, idx)


def _comb_body(r, main_ref, idx_ref, clse_ref, out_ref):
    z = main_ref[...].reshape(NHEAD, r)
    la3 = z[0:8, :].reshape(8, r // 128, 128)
    d2 = z[8:9, :].reshape(r // 128, 128)
    sg = idx_ref[...].reshape(r // 128, 128)
    clse = clse_ref[...]
    corr = jnp.zeros((r // 128, 128), jnp.float32)
    for b in range(NSEG):
        corr = jnp.where(sg == b, clse[b], corr)
    c2 = d2 - corr
    out_ref[...] = jnp.transpose(la3, (1, 0, 2)) + c2[:, None, :]


def _comb_call(main_flat, idx, clse, block_rows):
    n = idx.shape[0]
    t = n // 128
    bt = block_rows // 128
    return pl.pallas_call(
        functools.partial(_comb_body, block_rows),
        grid=(n // block_rows,),
        in_specs=[
            pl.BlockSpec((NHEAD * block_rows,), lambda i: (i,)),
            pl.BlockSpec((block_rows,), lambda i: (i,)),
            pl.BlockSpec((LANES,), lambda i: (0,)),
        ],
        out_specs=pl.BlockSpec((bt, 8, 128), lambda i: (i, 0, 0)),
        out_shape=jax.ShapeDtypeStruct((t, 8, 128), jnp.float32),
    )(main_flat, idx, clse)


def kernel(embed_states, batch_index, W_dev, b_dev, W_act, b_act):
    n, e = embed_states.shape
    a = W_act.shape[1]
    idx = batch_index.astype(jnp.int32)
    w_t = jnp.concatenate([W_act.T, W_dev.T], axis=0)
    b_t = jnp.concatenate([b_act, b_dev]).reshape(NHEAD, 1)
    main = _dense_call(embed_states, w_t, b_t, block_rows=4096)
    clse = _sc_call(main, idx, dense_block=4096)
    out3 = _comb_call(main, idx, clse, block_rows=4096)
    return out3.transpose(0, 2, 1).reshape(n, a)


# speedup vs baseline: 1.0457x; 1.0177x over previous
import functools

import jax
import jax.numpy as jnp
from jax import lax
from jax.experimental import pallas as pl
from jax.experimental.pallas import tpu as pltpu
from jax.experimental.pallas import tpu_sc as plsc

NUM_SUBCORES = 16
LANES = 16
NSEG = 16
NHEAD = 9
LN2 = 0.6931471805599453


def _dense_body(r, x_ref, wt_ref, bt_ref, out_ref):
    x = x_ref[...]
    yt = lax.dot_general(wt_ref[...], x, (((1,), (1,)), ((), ())),
                         preferred_element_type=jnp.float32)
    yt = yt + bt_ref[...]
    ya = yt[0:8, :]
    amax = jnp.max(ya, axis=0, keepdims=True)
    lse = jnp.log(jnp.sum(jnp.exp(ya - amax), axis=0, keepdims=True)) + amax
    z = jnp.concatenate([ya - lse, yt[8:9, :]], axis=0)
    out_ref[...] = z.reshape(NHEAD * r)


def _dense_call(x, w_t, b_t, block_rows):
    n, e = x.shape
    return pl.pallas_call(
        functools.partial(_dense_body, block_rows),
        grid=(n // block_rows,),
        in_specs=[
            pl.BlockSpec((block_rows, e), lambda i: (i, 0)),
            pl.BlockSpec((NHEAD, e), lambda i: (0, 0)),
            pl.BlockSpec((NHEAD, 1), lambda i: (0, 0)),
        ],
        out_specs=pl.BlockSpec((NHEAD * block_rows,), lambda i: (i,)),
        out_shape=jax.ShapeDtypeStruct((NHEAD * n,), jnp.float32),
    )(x, w_t, b_t)


def _sc_body(chunk, dense_block, main_hbm, idx_hbm, clse_hbm,
             dv, iv, lm_ref, s_ref, shared, part, clse_ref):
    minf = jnp.float32(-jnp.inf)
    wid = lax.axis_index("s")
    per = dense_block // chunk
    sbase = (wid // per) * NHEAD * dense_block + (wid % per) * chunk
    pltpu.sync_copy(main_hbm.at[pl.ds(sbase + 8 * dense_block, chunk)], dv)
    pltpu.sync_copy(idx_hbm.at[pl.ds(wid * chunk, chunk)], iv)

    nvec = chunk // LANES
    lane = lax.iota(jnp.int32, LANES)

    s_ref[...] = jnp.zeros((LANES,), jnp.float32)

    first = iv[pl.ds(0, LANES)][0]
    last = iv[pl.ds(chunk - LANES, LANES)][LANES - 1]

    @pl.when(first == last)
    def _chunk_uniform():
        def body(i, acc):
            return jnp.maximum(acc, dv[pl.ds(i * LANES, LANES)])

        acc = lax.fori_loop(0, nvec, body, jnp.full((LANES,), minf, jnp.float32),
                            unroll=8)
        m = jnp.max(acc)
        lm_ref[...] = jnp.where(lane == first, m, minf)

        def body2u(i, acc):
            return acc + jnp.exp(dv[pl.ds(i * LANES, LANES)] - m)

        acc2 = lax.fori_loop(0, nvec, body2u, jnp.zeros((LANES,), jnp.float32),
                             unroll=8)
        s_ref[...] = jnp.where(lane == first, jnp.sum(acc2), 0.0)

    @pl.when(first != last)
    def _chunk_mixed():
        def body(i, lm):
            v = dv[pl.ds(i * LANES, LANES)]
            sg = iv[pl.ds(i * LANES, LANES)]
            s0 = sg[0]
            s15 = sg[LANES - 1]

            def vec_uniform(lm):
                return jnp.where(lane == s0, jnp.maximum(lm, jnp.max(v)), lm)

            def vec_mixed(lm):
                def seg_loop(b, lm):
                    mb = jnp.max(jnp.where(sg == b, v, minf))
                    return jnp.where(lane == b, jnp.maximum(lm, mb), lm)

                return lax.fori_loop(0, NSEG, seg_loop, lm)

            return lax.cond(s0 == s15, vec_uniform, vec_mixed, lm)

        lm_ref[...] = lax.fori_loop(
            0, nvec, body, jnp.full((LANES,), minf, jnp.float32))

    @pl.when(first != last)
    def _chunk_mixed2():
        def body2(i, c):
            v = dv[pl.ds(i * LANES, LANES)]
            sg = iv[pl.ds(i * LANES, LANES)]
            shift = plsc.load_gather(lm_ref, [sg])
            plsc.addupdate_scatter(s_ref, [sg], jnp.exp(v - shift))
            return c

        lax.fori_loop(0, nvec, body2, 0, unroll=8)

    pltpu.sync_copy(lm_ref, shared.at[pl.ds(wid * 2 * LANES, LANES)])
    pltpu.sync_copy(s_ref, shared.at[pl.ds(wid * 2 * LANES + LANES, LANES)])
    plsc.subcore_barrier()

    @pl.when(wid == 0)
    def _reduce_and_emit():
        pltpu.sync_copy(shared, part)

        def mbody(w, m):
            return jnp.maximum(m, part[pl.ds(w * 2 * LANES, LANES)])

        m = lax.fori_loop(0, NUM_SUBCORES, mbody,
                          jnp.full((LANES,), minf, jnp.float32))

        def sbody(w, s):
            return s + part[pl.ds(w * 2 * LANES + LANES, LANES)] * jnp.exp(
                part[pl.ds(w * 2 * LANES, LANES)] - m)

        s = lax.fori_loop(0, NUM_SUBCORES, sbody,
                          jnp.zeros((LANES,), jnp.float32))

        bits = plsc.bitcast(s, jnp.int32)
        y = ((bits >> 23) - 127).astype(jnp.float32) * jnp.float32(LN2)
        for _ in range(4):
            y = y + s * jnp.exp(-y) - 1.0
        clse_ref[...] = m + y
        pltpu.sync_copy(clse_ref, clse_hbm)


def _sc_call(main_flat, idx, dense_block):
    n = idx.shape[0]
    chunk = n // NUM_SUBCORES
    mesh = plsc.VectorSubcoreMesh(
        core_axis_name="c", subcore_axis_name="s",
        num_cores=1, num_subcores=NUM_SUBCORES,
    )
    return pl.kernel(
        functools.partial(_sc_body, chunk, dense_block),
        out_type=jax.ShapeDtypeStruct((LANES,), jnp.float32),
        mesh=mesh,
        compiler_params=pltpu.CompilerParams(needs_layout_passes=False),
        scratch_types=[
            pltpu.VMEM((chunk,), jnp.float32),
            pltpu.VMEM((chunk,), jnp.int32),
            pltpu.VMEM((LANES,), jnp.float32),
            pltpu.VMEM((LANES,), jnp.float32),
            pltpu.VMEM_SHARED((NUM_SUBCORES * 2 * LANES,), jnp.float32),
            pltpu.VMEM((NUM_SUBCORES * 2 * LANES,), jnp.float32),
            pltpu.VMEM((LANES,), jnp.float32),
        ],
    )(main_flat, idx)


def _comb_body(r, main_ref, idx_ref, clse_ref, out_ref):
    t = r // 128
    z = main_ref[...].reshape(NHEAD * t, 128)
    la3 = z[0:8 * t, :].reshape(8, t, 128)
    d2 = z[8 * t:NHEAD * t, :]
    sg = idx_ref[...].reshape(t, 128)
    clse = clse_ref[...]
    corr = jnp.zeros((t, 128), jnp.float32)
    for b in range(NSEG):
        corr = jnp.where(sg == b, clse[b], corr)
    c2 = d2 - corr
    out_ref[...] = jnp.transpose(la3, (1, 0, 2)) + c2[:, None, :]


def _comb_call(main_flat, idx, clse, block_rows):
    n = idx.shape[0]
    t = n // 128
    bt = block_rows // 128
    return pl.pallas_call(
        functools.partial(_comb_body, block_rows),
        grid=(n // block_rows,),
        in_specs=[
            pl.BlockSpec((NHEAD * block_rows,), lambda i: (i,)),
            pl.BlockSpec((block_rows,), lambda i: (i,)),
            pl.BlockSpec((LANES,), lambda i: (0,)),
        ],
        out_specs=pl.BlockSpec((bt, 8, 128), lambda i: (i, 0, 0)),
        out_shape=jax.ShapeDtypeStruct((t, 8, 128), jnp.float32),
    )(main_flat, idx, clse)


def kernel(embed_states, batch_index, W_dev, b_dev, W_act, b_act):
    n, e = embed_states.shape
    a = W_act.shape[1]
    idx = batch_index.astype(jnp.int32)
    w_t = jnp.concatenate([W_act.T, W_dev.T], axis=0)
    b_t = jnp.concatenate([b_act, b_dev]).reshape(NHEAD, 1)
    main = _dense_call(embed_states, w_t, b_t, block_rows=4096)
    clse = _sc_call(main, idx, dense_block=4096)
    out3 = _comb_call(main, idx, clse, block_rows=4096)
    return out3.transpose(0, 2, 1).reshape(n, a)
